# baseline (speedup 1.0000x reference)
import jax
import jax.numpy as jnp
from jax import lax
from jax.experimental import pallas as pl
from jax.experimental.pallas import tpu as pltpu
from jax.experimental.pallas import tpu_sc as plsc

NQ = 16
E = 1600000
NN = 100000
DF = 128
KSEL = 1 + (128 - 1) * (128 - 2) // 2
NC, NS, NW = 2, 16, 32
EPH = E // 2
ROWS = 102400
MARKS = 8 * ROWS
DUMMY = 7 * ROWS + 101000
NBUCK = 2048

_mesh = plsc.VectorSubcoreMesh(
    core_axis_name="c", subcore_axis_name="s", num_cores=NC, num_subcores=NS)

_i32 = jnp.int32
_u32 = jnp.uint32
_STOP = 0


def _iota16():
    return lax.iota(_i32, 16)


def _wid():
    return lax.axis_index("s") * NC + lax.axis_index("c")


_KWIN = 3200
_KNWIN = EPH // _KWIN


def _keygen_body(ents_hbm, cons_hbm, src_hbm, dst_hbm, w_hbm, keys_hbm,
                 boost_v, ents_v, cons_v, src_v0, src_v1, dst_v0, dst_v1,
                 w_v0, w_v1, key_v0, key_v1, sin0, sin1, sout0, sout1):
    wid = _wid()
    b = wid // 2
    base = (wid % 2) * EPH
    srcb = (src_v0, src_v1)
    dstb = (dst_v0, dst_v1)
    wb = (w_v0, w_v1)
    keyb = (key_v0, key_v1)
    sins = (sin0, sin1)
    souts = (sout0, sout1)
    WIN = _KWIN

    zf = jnp.zeros((16,), jnp.float32)

    def zero_b(i, _):
        boost_v[pl.ds(i * 16, 16)] = zf
        return _
    lax.fori_loop(0, NN // 16, zero_b, None)

    pltpu.sync_copy(ents_hbm.at[pl.ds(b * 32, 32)], ents_v)
    pltpu.sync_copy(cons_hbm.at[pl.ds(b * 32, 32)], cons_v)
    lane = _iota16()
    for g in range(2):
        ev = ents_v[pl.ds(g * 16, 16)]
        cv = cons_v[pl.ds(g * 16, 16)]
        for j in range(16):
            plsc.addupdate_scatter(boost_v, [ev], cv, mask=(lane == j))

    for bufi in range(2):
        off = base + bufi * WIN
        pltpu.async_copy(src_hbm.at[pl.ds(off, WIN)], srcb[bufi], sins[bufi])
        pltpu.async_copy(dst_hbm.at[pl.ds(off, WIN)], dstb[bufi], sins[bufi])
        pltpu.async_copy(w_hbm.at[pl.ds(off, WIN)], wb[bufi], sins[bufi])

    def outer(g, _):
        for bufi in range(2):
            widx = g * 2 + bufi
            off = base + widx * WIN
            pltpu.make_async_copy(src_hbm.at[pl.ds(off, WIN)],
                                  srcb[bufi], sins[bufi]).wait()
            pltpu.make_async_copy(dst_hbm.at[pl.ds(off, WIN)],
                                  dstb[bufi], sins[bufi]).wait()
            pltpu.make_async_copy(w_hbm.at[pl.ds(off, WIN)],
                                  wb[bufi], sins[bufi]).wait()

            @pl.when(widx >= 2)
            def _wait_out():
                pltpu.make_async_copy(
                    keyb[bufi], keys_hbm.at[pl.ds(b * E + off, WIN)],
                    souts[bufi]).wait()

            def inner(i, _):
                for j in range(5):
                    pos = i * 80 + j * 16
                    s16 = srcb[bufi][pl.ds(pos, 16)]
                    d16 = dstb[bufi][pl.ds(pos, 16)]
                    wv = wb[bufi][pl.ds(pos, 16)]
                    val = wv + plsc.load_gather(boost_v, [s16]) \
                        + plsc.load_gather(boost_v, [d16])
                    keyb[bufi][pl.ds(pos, 16)] = plsc.bitcast(val, _u32)
                return _
            lax.fori_loop(0, WIN // 80, inner, None)
            pltpu.async_copy(keyb[bufi],
                             keys_hbm.at[pl.ds(b * E + off, WIN)],
                             souts[bufi])

            @pl.when(widx + 2 < _KNWIN)
            def _prefetch():
                noff = off + 2 * WIN
                pltpu.async_copy(src_hbm.at[pl.ds(noff, WIN)],
                                 srcb[bufi], sins[bufi])
                pltpu.async_copy(dst_hbm.at[pl.ds(noff, WIN)],
                                 dstb[bufi], sins[bufi])
                pltpu.async_copy(w_hbm.at[pl.ds(noff, WIN)],
                                 wb[bufi], sins[bufi])
        return _
    lax.fori_loop(0, _KNWIN // 2, outer, None)
    for bufi in range(2):
        pltpu.make_async_copy(keyb[bufi],
                              keys_hbm.at[pl.ds(b * E + base, WIN)],
                              souts[bufi]).wait()


def _keygen(ents, cons, src, dst, w):
    f = pl.kernel(
        _keygen_body,
        out_type=jax.ShapeDtypeStruct((NQ * E,), _u32),
        mesh=_mesh,
        compiler_params=pltpu.CompilerParams(needs_layout_passes=False),
        scratch_types=[
            pltpu.VMEM((NN,), jnp.float32),
            pltpu.VMEM((32,), _i32),
            pltpu.VMEM((32,), jnp.float32),
            pltpu.VMEM((_KWIN,), _i32),
            pltpu.VMEM((_KWIN,), _i32),
            pltpu.VMEM((_KWIN,), _i32),
            pltpu.VMEM((_KWIN,), _i32),
            pltpu.VMEM((_KWIN,), jnp.float32),
            pltpu.VMEM((_KWIN,), jnp.float32),
            pltpu.VMEM((_KWIN,), _u32),
            pltpu.VMEM((_KWIN,), _u32),
            pltpu.SemaphoreType.DMA,
            pltpu.SemaphoreType.DMA,
            pltpu.SemaphoreType.DMA,
            pltpu.SemaphoreType.DMA,
        ],
    )
    return f(ents, cons, src, dst, w)


def _make_hist(bshift, bmask, from_index, mshift, use_m2, m2shift):
    WIN = 4000
    NWIN = EPH // WIN

    def body(keys_hbm, mval_hbm, m2val_hbm, hist_hbm, keys_v0, keys_v1,
             mval_v, m2val_v, hist_v, sem0, sem1):
        wid = _wid()
        b = wid // 2
        base = (wid % 2) * EPH
        zi = jnp.zeros((16,), _i32)
        sems = (sem0, sem1)
        bufs = (keys_v0, keys_v1)

        def zero_h(i, _):
            hist_v[pl.ds(i * 16, 16)] = zi
            return _
        lax.fori_loop(0, 32 * NBUCK // 16, zero_h, None)

        pltpu.sync_copy(mval_hbm.at[pl.ds(b * 16, 16)], mval_v)
        pltpu.sync_copy(m2val_hbm.at[pl.ds(b * 16, 16)], m2val_v)
        mv = mval_v[...]
        m2v = m2val_v[...]
        lane = _iota16()
        ones = jnp.ones((16,), _i32)

        for bufi in range(2):
            pltpu.async_copy(
                keys_hbm.at[pl.ds(b * E + base + bufi * WIN, WIN)],
                bufs[bufi], sems[bufi])

        def outer(g, _):
            for bufi in range(2):
                widx = g * 2 + bufi
                off = base + widx * WIN
                pltpu.make_async_copy(
                    keys_hbm.at[pl.ds(b * E + off, WIN)],
                    bufs[bufi], sems[bufi]).wait()

                def inner(i, _):
                    for j in range(10):
                        pos = i * 160 + j * 16
                        k16 = bufs[bufi][pl.ds(pos, 16)]
                        gi16 = off + pos + lane
                        if mshift is None:
                            matched = None
                        else:
                            matched = (k16 >> _u32(mshift)) == mv
                        if use_m2:
                            matched = matched & ((gi16 >> m2shift) == m2v)
                        if from_index:
                            bucket = (gi16 >> bshift) & bmask
                        elif bshift == 20:
                            bucket = (k16 >> _u32(20)).astype(_i32)
                        else:
                            bucket = ((k16 >> _u32(bshift))
                                      & _u32(bmask)).astype(_i32)
                        plsc.addupdate_scatter(
                            hist_v, [(j % 2) * 16 * NBUCK
                                     + lane * NBUCK + bucket], ones,
                            mask=matched)
                    return _
                lax.fori_loop(0, WIN // 160, inner, None)

                @pl.when(widx + 2 < NWIN)
                def _prefetch():
                    pltpu.async_copy(
                        keys_hbm.at[pl.ds(b * E + off + 2 * WIN, WIN)],
                        bufs[bufi], sems[bufi])
            return _
        lax.fori_loop(0, NWIN // 2, outer, None)
        pltpu.sync_copy(hist_v, hist_hbm.at[pl.ds(wid * 32 * NBUCK,
                                                  32 * NBUCK)])

    def run(keys, mval, m2val):
        f = pl.kernel(
            body,
            out_type=jax.ShapeDtypeStruct((NW * 32 * NBUCK,), _i32),
            mesh=_mesh,
            compiler_params=pltpu.CompilerParams(needs_layout_passes=False),
            scratch_types=[
                pltpu.VMEM((WIN,), _u32),
                pltpu.VMEM((WIN,), _u32),
                pltpu.VMEM((16,), _u32),
                pltpu.VMEM((16,), _i32),
                pltpu.VMEM((32 * NBUCK,), _i32),
                pltpu.SemaphoreType.DMA,
                pltpu.SemaphoreType.DMA,
            ],
        )
        raw = f(keys, mval, m2val)
        return raw.reshape(NQ, 2, 32, NBUCK).sum(axis=(1, 2))
    return run


_hist_p1 = _make_hist(20, 2047, False, None, False, 0)
_hist_p2 = _make_hist(9, 2047, False, 20, False, 0)
_hist_p3 = _make_hist(0, 511, False, 9, False, 0)
_hist_tA = _make_hist(10, 2047, True, 0, False, 0)
_hist_tB = _make_hist(0, 1023, True, 0, True, 10)


def _pick_desc(h, k):
    s = jnp.cumsum(h[:, ::-1], axis=1)[:, ::-1]
    i = jnp.sum((s >= k[:, None]).astype(_i32), axis=1) - 1
    s_next = jnp.concatenate([s[:, 1:], jnp.zeros((NQ, 1), s.dtype)], axis=1)
    above = jnp.take_along_axis(s_next, i[:, None], axis=1)[:, 0]
    return i, k - above


def _pick_asc(h, r):
    p = jnp.cumsum(h, axis=1)
    i = jnp.sum((p < r[:, None]).astype(_i32), axis=1)
    p_excl = p - h
    r_next = r - jnp.take_along_axis(p_excl, i[:, None], axis=1)[:, 0]
    return i, r_next


def _mark_body(keys_hbm, src_hbm, dst_hbm, tval_hbm, ibnd_hbm, nodes_hbm,
               marks_sh, coll_sh, cnts_sh,
               src_v, dst_v, key_v0, key_v1, sidx_v0, sidx_v1, sval_v0,
               sval_v1, z2k_v, tv_all, ib_all,
               seg_v, ids_v, cnt_v, call_v, coll_v, sb_v, out_v,
               skin0, skin1, sscat0, sscat1):
    c = lax.axis_index("c")
    s = lax.axis_index("s")
    lane = _iota16()
    keyb = (key_v0, key_v1)
    sidxb = (sidx_v0, sidx_v1)
    svalb = (sval_v0, sval_v1)
    skins = (skin0, skin1)
    sscats = (sscat0, sscat1)

    zi = jnp.zeros((16,), _i32)

    def zero_z(i, _):
        z2k_v[pl.ds(i * 16, 16)] = zi
        return _
    lax.fori_loop(0, 128, zero_z, None)

    def zero_m(i, _):
        pltpu.sync_copy(z2k_v, marks_sh.at[pl.ds(s * 51200 + i * 2048, 2048)])
        return _
    lax.fori_loop(0, 25, zero_m, None)
    plsc.subcore_barrier()

    pltpu.sync_copy(tval_hbm.at[pl.ds(c * 8 * 16, 128)], tv_all)
    pltpu.sync_copy(ibnd_hbm.at[pl.ds(c * 8 * 16, 128)], ib_all)

    for p in range(2):
        for t in range(6):
            sidxb[p][pl.ds(4000 + t * 16, 16)] = jnp.full((16,), DUMMY,
                                                          _i32)
            svalb[p][pl.ds(4000 + t * 16, 16)] = zi

    WIN = 2000
    NWINM = NN // WIN

    for p in range(2):
        pltpu.async_copy(
            keys_hbm.at[pl.ds((c * 8 + p) * E + s * NN, WIN)],
            keyb[p], skins[p])

    def win(wi, _):
        eoff = s * NN + wi * WIN
        pltpu.sync_copy(src_hbm.at[pl.ds(eoff, WIN)], src_v)
        pltpu.sync_copy(dst_hbm.at[pl.ds(eoff, WIN)], dst_v)
        for bl in range(8):
            step = wi * 8 + bl
            p = bl % 2
            bg = c * 8 + bl
            pltpu.make_async_copy(
                keys_hbm.at[pl.ds(bg * E + eoff, WIN)],
                keyb[p], skins[p]).wait()

            @pl.when(step >= 2)
            def _wait_scat():
                pltpu.make_async_copy(svalb[p], marks_sh.at[sidxb[p]],
                                      sscats[p]).wait()
            tv = tv_all[pl.ds(bl * 16, 16)]
            iv = ib_all[pl.ds(bl * 16, 16)]

            def inner(i, _):
                for j in range(5):
                    pos = i * 80 + j * 16
                    k16 = keyb[p][pl.ds(pos, 16)]
                    s16 = src_v[pl.ds(pos, 16)]
                    d16 = dst_v[pl.ds(pos, 16)]
                    gi16 = eoff + pos + lane
                    sel = (k16 > tv) | ((k16 == tv) & (gi16 <= iv))
                    val = sel.astype(_i32)
                    sidxb[p][pl.ds(pos, 16)] = bl * ROWS + s16
                    svalb[p][pl.ds(pos, 16)] = val
                    sidxb[p][pl.ds(2000 + pos, 16)] = bl * ROWS + d16
                    svalb[p][pl.ds(2000 + pos, 16)] = val
                return _
            lax.fori_loop(0, WIN // 80, inner, None)
            pltpu.async_copy(svalb[p], marks_sh.at[sidxb[p]], sscats[p],
                             add=True)

            bl2 = (bl + 2) % 8
            wi2 = wi + (bl + 2) // 8
            bg2 = c * 8 + bl2

            @pl.when(wi2 < NWINM)
            def _prefetch():
                eoff2 = s * NN + wi2 * WIN
                pltpu.async_copy(
                    keys_hbm.at[pl.ds(bg2 * E + eoff2, WIN)],
                    keyb[p], skins[p])
        return _
    lax.fori_loop(0, NWINM, win, None)
    for p in range(2):
        pltpu.make_async_copy(svalb[p], marks_sh.at[sidxb[p]],
                              sscats[p]).wait()
    plsc.subcore_barrier()

    cnts = jnp.zeros((16,), _i32)
    for bl in range(8):
        pltpu.sync_copy(
            marks_sh.at[pl.ds(bl * ROWS + s * 6400, 6400)], seg_v)

        def scan(i, ptr):
            m16 = seg_v[pl.ds(i * 16, 16)] > 0
            gid16 = s * 6400 + i * 16 + lane
            m16 = m16 & (gid16 < NN)
            cnt = jnp.sum(m16.astype(_i32))

            @pl.when(ptr < 128)
            def _store():
                plsc.store_compressed(ids_v.at[pl.ds(ptr, 16)], gid16,
                                      mask=m16)
            return ptr + cnt
        ptr = lax.fori_loop(0, 400, scan, _i32(0))
        cnts = jnp.where(lane == bl, ptr, cnts)
        pltpu.sync_copy(ids_v, coll_sh.at[pl.ds((s * 8 + bl) * 160, 160)])
    cnt_v[...] = cnts
    pltpu.sync_copy(cnt_v, cnts_sh.at[pl.ds(s * 16, 16)])
    plsc.subcore_barrier()

    @pl.when(s < 8)
    def _assemble():
        pltpu.sync_copy(cnts_sh, call_v)
        for seg in range(16):
            pltpu.sync_copy(coll_sh.at[pl.ds((seg * 8 + s) * 160, 160)],
                            coll_v.at[pl.ds(seg * 160, 160)])
        counts16 = plsc.load_gather(call_v, [lane * 16 + s])
        capped = jnp.minimum(counts16, 128)
        exclc = plsc.cumsum(capped) - capped
        take = jnp.clip(128 - exclc, 0, capped)
        opos = plsc.cumsum(take) - take
        bound = plsc.cumsum(take)
        total = jnp.sum(take)
        sb_v[pl.ds(0, 16)] = bound
        sb_v[pl.ds(16, 16)] = opos

        for j in range(8):
            p16 = j * 16 + lane
            segidx = jnp.zeros((16,), _i32)
            for t in range(16):
                bt = plsc.load_gather(sb_v, [jnp.full((16,), t, _i32)])
                segidx = segidx + (bt <= p16).astype(_i32)
            segidx = jnp.minimum(segidx, 15)
            op = plsc.load_gather(sb_v, [16 + segidx])
            addr = segidx * 160 + (p16 - op)
            ids16 = plsc.load_gather(coll_v, [addr])
            out_v[pl.ds(j * 16, 16)] = jnp.where(p16 < total, ids16, 0)
        bg = c * 8 + s
        pltpu.sync_copy(out_v, nodes_hbm.at[pl.ds(bg * 128, 128)])


def _mark_scan(keys, src, dst, tval, ibnd):
    f = pl.kernel(
        _mark_body,
        out_type=jax.ShapeDtypeStruct((NQ * 128,), _i32),
        mesh=_mesh,
        compiler_params=pltpu.CompilerParams(needs_layout_passes=False),
        scratch_types=[
            pltpu.VMEM_SHARED((MARKS,), _i32),
            pltpu.VMEM_SHARED((16 * 8 * 160,), _i32),
            pltpu.VMEM_SHARED((256,), _i32),
            pltpu.VMEM((2000,), _i32),
            pltpu.VMEM((2000,), _i32),
            pltpu.VMEM((2000,), _u32),
            pltpu.VMEM((2000,), _u32),
            pltpu.VMEM((4096,), _i32),
            pltpu.VMEM((4096,), _i32),
            pltpu.VMEM((4096,), _i32),
            pltpu.VMEM((4096,), _i32),
            pltpu.VMEM((2048,), _i32),
            pltpu.VMEM((128,), _u32),
            pltpu.VMEM((128,), _i32),
            pltpu.VMEM((6400,), _i32),
            pltpu.VMEM((160,), _i32),
            pltpu.VMEM((16,), _i32),
            pltpu.VMEM((256,), _i32),
            pltpu.VMEM((16 * 160,), _i32),
            pltpu.VMEM((32,), _i32),
            pltpu.VMEM((128,), _i32),
            pltpu.SemaphoreType.DMA,
            pltpu.SemaphoreType.DMA,
            pltpu.SemaphoreType.DMA,
            pltpu.SemaphoreType.DMA,
        ],
    )
    return f(keys, src, dst, tval, ibnd)


def _gather_body(table_hbm, idx_hbm, out_hbm, idx_v, rows_v, sem):
    wid = _wid()
    base = wid * 64
    pltpu.sync_copy(idx_hbm.at[pl.ds(base, 64)], idx_v)
    pltpu.async_copy(table_hbm.at[idx_v], rows_v, sem).wait()
    pltpu.sync_copy(rows_v, out_hbm.at[pl.ds(base, 64)])


def _gather_rows(table, idx):
    f = pl.kernel(
        _gather_body,
        out_type=jax.ShapeDtypeStruct((NQ * 128, DF), jnp.float32),
        mesh=_mesh,
        compiler_params=pltpu.CompilerParams(needs_layout_passes=False),
        scratch_types=[
            pltpu.VMEM((64,), _i32),
            pltpu.VMEM((64, DF), jnp.float32),
            pltpu.SemaphoreType.DMA,
        ],
    )
    return f(table, idx)


def _rep16(x, dtype):
    return jnp.tile(x.astype(dtype)[:, None], (1, 16)).reshape(-1)


@jax.jit
def _run(attention_question, question_entities, edge_index, edge_weights,
         node_table, w_imp, num_max_nodes):
    importance = jax.nn.sigmoid(attention_question * w_imp)
    contrib = importance * (importance >= 0.5).astype(importance.dtype)

    ents = jnp.pad(question_entities, ((0, 0), (0, 12))).reshape(-1)
    cons = jnp.pad(contrib, ((0, 0), (0, 12))).reshape(-1)
    src = edge_index[0]
    dst = edge_index[1]

    keys = _keygen(ents, cons, src, dst, edge_weights)
    if _STOP == 1:
        return keys[:NQ * 128 * DF].astype(jnp.float32).reshape(NQ, 128, DF)

    zero16 = jnp.zeros((NQ * 16,), _i32)
    k1 = jnp.full((NQ,), KSEL, _i32)
    h1 = _hist_p1(keys, _rep16(jnp.zeros((NQ,), _u32), _u32), zero16)
    i1, k2 = _pick_desc(h1, k1)
    h2 = _hist_p2(keys, _rep16(i1, _u32), zero16)
    i2, k3 = _pick_desc(h2, k2)
    h3 = _hist_p3(keys, _rep16((i1 << 11) | i2, _u32), zero16)
    i3, r = _pick_desc(h3, k3)
    tval = ((i1.astype(_u32) << 20) | (i2.astype(_u32) << 9)
            | i3.astype(_u32))
    ha = _hist_tA(keys, _rep16(tval, _u32), zero16)
    ia, rb = _pick_asc(ha, r)
    hb = _hist_tB(keys, _rep16(tval, _u32), _rep16(ia, _i32))
    ib, _ = _pick_asc(hb, rb)
    ibnd = ia * 1024 + ib
    if _STOP == 2:
        return (jnp.zeros((NQ, 128, DF), jnp.float32)
                + (tval.sum() + ibnd.sum()).astype(jnp.float32))

    nodes = _mark_scan(keys, src, dst, _rep16(tval, _u32),
                       _rep16(ibnd, _i32))
    nodes = nodes + (jnp.asarray(num_max_nodes, _i32) - 128)
    out = _gather_rows(node_table, nodes)
    return out.reshape(NQ, 128, DF)


def kernel(attention_question, question_entities, edge_index, edge_weights,
           node_table, w_imp, num_max_nodes):
    return _run(attention_question, question_entities, edge_index,
                edge_weights, node_table, w_imp, num_max_nodes)


# speedup vs baseline: 1.0481x; 1.0481x over previous
import jax
import jax.numpy as jnp
from jax import lax
from jax.experimental import pallas as pl
from jax.experimental.pallas import tpu as pltpu
from jax.experimental.pallas import tpu_sc as plsc

NQ = 16
E = 1600000
NN = 100000
DF = 128
KSEL = 1 + (128 - 1) * (128 - 2) // 2
NC, NS, NW = 2, 16, 32
EPH = E // 2
ROWS = 102400
MARKS = 8 * ROWS
DUMMY = 7 * ROWS + 101000
NBUCK = 2048

_mesh = plsc.VectorSubcoreMesh(
    core_axis_name="c", subcore_axis_name="s", num_cores=NC, num_subcores=NS)

_i32 = jnp.int32
_u32 = jnp.uint32
_STOP = 0


def _iota16():
    return lax.iota(_i32, 16)


def _wid():
    return lax.axis_index("s") * NC + lax.axis_index("c")


_KWIN = 3200
_KNWIN = EPH // _KWIN


def _keygen_body(ents_hbm, cons_hbm, src_hbm, dst_hbm, w_hbm, keys_hbm,
                 boost_v, ents_v, cons_v, src_v0, src_v1, dst_v0, dst_v1,
                 w_v0, w_v1, key_v0, key_v1, sin0, sin1, sout0, sout1):
    wid = _wid()
    b = wid // 2
    base = (wid % 2) * EPH
    srcb = (src_v0, src_v1)
    dstb = (dst_v0, dst_v1)
    wb = (w_v0, w_v1)
    keyb = (key_v0, key_v1)
    sins = (sin0, sin1)
    souts = (sout0, sout1)
    WIN = _KWIN

    zf = jnp.zeros((16,), jnp.float32)

    def zero_b(i, _):
        boost_v[pl.ds(i * 16, 16)] = zf
        return _
    lax.fori_loop(0, NN // 16, zero_b, None)

    pltpu.sync_copy(ents_hbm.at[pl.ds(b * 32, 32)], ents_v)
    pltpu.sync_copy(cons_hbm.at[pl.ds(b * 32, 32)], cons_v)
    lane = _iota16()
    for g in range(2):
        ev = ents_v[pl.ds(g * 16, 16)]
        cv = cons_v[pl.ds(g * 16, 16)]
        for j in range(16):
            plsc.addupdate_scatter(boost_v, [ev], cv, mask=(lane == j))

    for bufi in range(2):
        off = base + bufi * WIN
        pltpu.async_copy(src_hbm.at[pl.ds(off, WIN)], srcb[bufi], sins[bufi])
        pltpu.async_copy(dst_hbm.at[pl.ds(off, WIN)], dstb[bufi], sins[bufi])
        pltpu.async_copy(w_hbm.at[pl.ds(off, WIN)], wb[bufi], sins[bufi])

    def outer(g, _):
        for bufi in range(2):
            widx = g * 2 + bufi
            off = base + widx * WIN
            pltpu.make_async_copy(src_hbm.at[pl.ds(off, WIN)],
                                  srcb[bufi], sins[bufi]).wait()
            pltpu.make_async_copy(dst_hbm.at[pl.ds(off, WIN)],
                                  dstb[bufi], sins[bufi]).wait()
            pltpu.make_async_copy(w_hbm.at[pl.ds(off, WIN)],
                                  wb[bufi], sins[bufi]).wait()

            @pl.when(widx >= 2)
            def _wait_out():
                pltpu.make_async_copy(
                    keyb[bufi], keys_hbm.at[pl.ds(b * E + off, WIN)],
                    souts[bufi]).wait()

            def inner(i, _):
                for j in range(5):
                    pos = i * 80 + j * 16
                    s16 = srcb[bufi][pl.ds(pos, 16)]
                    d16 = dstb[bufi][pl.ds(pos, 16)]
                    wv = wb[bufi][pl.ds(pos, 16)]
                    val = wv + plsc.load_gather(boost_v, [s16]) \
                        + plsc.load_gather(boost_v, [d16])
                    keyb[bufi][pl.ds(pos, 16)] = plsc.bitcast(val, _u32)
                return _
            lax.fori_loop(0, WIN // 80, inner, None)
            pltpu.async_copy(keyb[bufi],
                             keys_hbm.at[pl.ds(b * E + off, WIN)],
                             souts[bufi])

            @pl.when(widx + 2 < _KNWIN)
            def _prefetch():
                noff = off + 2 * WIN
                pltpu.async_copy(src_hbm.at[pl.ds(noff, WIN)],
                                 srcb[bufi], sins[bufi])
                pltpu.async_copy(dst_hbm.at[pl.ds(noff, WIN)],
                                 dstb[bufi], sins[bufi])
                pltpu.async_copy(w_hbm.at[pl.ds(noff, WIN)],
                                 wb[bufi], sins[bufi])
        return _
    lax.fori_loop(0, _KNWIN // 2, outer, None)
    for bufi in range(2):
        pltpu.make_async_copy(keyb[bufi],
                              keys_hbm.at[pl.ds(b * E + base, WIN)],
                              souts[bufi]).wait()


def _keygen(ents, cons, src, dst, w):
    f = pl.kernel(
        _keygen_body,
        out_type=jax.ShapeDtypeStruct((NQ * E,), _u32),
        mesh=_mesh,
        compiler_params=pltpu.CompilerParams(needs_layout_passes=False),
        scratch_types=[
            pltpu.VMEM((NN,), jnp.float32),
            pltpu.VMEM((32,), _i32),
            pltpu.VMEM((32,), jnp.float32),
            pltpu.VMEM((_KWIN,), _i32),
            pltpu.VMEM((_KWIN,), _i32),
            pltpu.VMEM((_KWIN,), _i32),
            pltpu.VMEM((_KWIN,), _i32),
            pltpu.VMEM((_KWIN,), jnp.float32),
            pltpu.VMEM((_KWIN,), jnp.float32),
            pltpu.VMEM((_KWIN,), _u32),
            pltpu.VMEM((_KWIN,), _u32),
            pltpu.SemaphoreType.DMA,
            pltpu.SemaphoreType.DMA,
            pltpu.SemaphoreType.DMA,
            pltpu.SemaphoreType.DMA,
        ],
    )
    return f(ents, cons, src, dst, w)


def _make_hist(bshift, bmask, from_index, mshift, use_m2, m2shift):
    WIN = 4000
    NWIN = EPH // WIN

    def body(keys_hbm, mval_hbm, m2val_hbm, hist_hbm, keys_v0, keys_v1,
             mval_v, m2val_v, hist_v, sem0, sem1):
        wid = _wid()
        b = wid // 2
        base = (wid % 2) * EPH
        zi = jnp.zeros((16,), _i32)
        sems = (sem0, sem1)
        bufs = (keys_v0, keys_v1)

        def zero_h(i, _):
            hist_v[pl.ds(i * 16, 16)] = zi
            return _
        lax.fori_loop(0, 16 * (NBUCK + 1) // 16, zero_h, None)

        pltpu.sync_copy(mval_hbm.at[pl.ds(b * 16, 16)], mval_v)
        pltpu.sync_copy(m2val_hbm.at[pl.ds(b * 16, 16)], m2val_v)
        mv = mval_v[...]
        m2v = m2val_v[...]
        lane = _iota16()
        ones = jnp.ones((16,), _i32)

        for bufi in range(2):
            pltpu.async_copy(
                keys_hbm.at[pl.ds(b * E + base + bufi * WIN, WIN)],
                bufs[bufi], sems[bufi])

        def outer(g, _):
            for bufi in range(2):
                widx = g * 2 + bufi
                off = base + widx * WIN
                pltpu.make_async_copy(
                    keys_hbm.at[pl.ds(b * E + off, WIN)],
                    bufs[bufi], sems[bufi]).wait()

                def inner(i, _):
                    for j in range(10):
                        pos = i * 160 + j * 16
                        k16 = bufs[bufi][pl.ds(pos, 16)]
                        gi16 = off + pos + lane
                        if mshift is None:
                            matched = None
                        else:
                            matched = (k16 >> _u32(mshift)) == mv
                        if use_m2:
                            matched = matched & ((gi16 >> m2shift) == m2v)
                        if from_index:
                            bucket = (gi16 >> bshift) & bmask
                        elif bshift == 20:
                            bucket = (k16 >> _u32(20)).astype(_i32)
                        else:
                            bucket = ((k16 >> _u32(bshift))
                                      & _u32(bmask)).astype(_i32)
                        plsc.addupdate_scatter(
                            hist_v, [lane * (NBUCK + 1) + bucket], ones,
                            mask=matched)
                    return _
                lax.fori_loop(0, WIN // 160, inner, None)

                @pl.when(widx + 2 < NWIN)
                def _prefetch():
                    pltpu.async_copy(
                        keys_hbm.at[pl.ds(b * E + off + 2 * WIN, WIN)],
                        bufs[bufi], sems[bufi])
            return _
        lax.fori_loop(0, NWIN // 2, outer, None)
        pltpu.sync_copy(hist_v, hist_hbm.at[pl.ds(wid * 16 * (NBUCK + 1),
                                                  16 * (NBUCK + 1))])

    def run(keys, mval, m2val):
        f = pl.kernel(
            body,
            out_type=jax.ShapeDtypeStruct((NW * 16 * (NBUCK + 1),), _i32),
            mesh=_mesh,
            compiler_params=pltpu.CompilerParams(needs_layout_passes=False),
            scratch_types=[
                pltpu.VMEM((WIN,), _u32),
                pltpu.VMEM((WIN,), _u32),
                pltpu.VMEM((16,), _u32),
                pltpu.VMEM((16,), _i32),
                pltpu.VMEM((16 * (NBUCK + 1),), _i32),
                pltpu.SemaphoreType.DMA,
                pltpu.SemaphoreType.DMA,
            ],
        )
        raw = f(keys, mval, m2val)
        raw = raw.reshape(NQ, 2, 16, NBUCK + 1)[..., :NBUCK]
        return raw.sum(axis=(1, 2))
    return run


_hist_p1 = _make_hist(20, 2047, False, None, False, 0)
_hist_p2 = _make_hist(9, 2047, False, 20, False, 0)
_hist_p3 = _make_hist(0, 511, False, 9, False, 0)
_hist_tA = _make_hist(10, 2047, True, 0, False, 0)
_hist_tB = _make_hist(0, 1023, True, 0, True, 10)


def _pick_desc(h, k):
    s = jnp.cumsum(h[:, ::-1], axis=1)[:, ::-1]
    i = jnp.sum((s >= k[:, None]).astype(_i32), axis=1) - 1
    s_next = jnp.concatenate([s[:, 1:], jnp.zeros((NQ, 1), s.dtype)], axis=1)
    above = jnp.take_along_axis(s_next, i[:, None], axis=1)[:, 0]
    return i, k - above


def _pick_asc(h, r):
    p = jnp.cumsum(h, axis=1)
    i = jnp.sum((p < r[:, None]).astype(_i32), axis=1)
    p_excl = p - h
    r_next = r - jnp.take_along_axis(p_excl, i[:, None], axis=1)[:, 0]
    return i, r_next


def _mark_body(keys_hbm, src_hbm, dst_hbm, tval_hbm, ibnd_hbm, nodes_hbm,
               marks_sh, coll_sh, cnts_sh,
               src_v, dst_v, key_v0, key_v1, sidx_v0, sidx_v1, sval_v0,
               sval_v1, z2k_v, tv_all, ib_all,
               seg_v, ids_v, cnt_v, call_v, coll_v, sb_v, out_v,
               skin0, skin1, sscat0, sscat1):
    c = lax.axis_index("c")
    s = lax.axis_index("s")
    lane = _iota16()
    keyb = (key_v0, key_v1)
    sidxb = (sidx_v0, sidx_v1)
    svalb = (sval_v0, sval_v1)
    skins = (skin0, skin1)
    sscats = (sscat0, sscat1)

    zi = jnp.zeros((16,), _i32)

    def zero_z(i, _):
        z2k_v[pl.ds(i * 16, 16)] = zi
        return _
    lax.fori_loop(0, 128, zero_z, None)

    def zero_m(i, _):
        pltpu.sync_copy(z2k_v, marks_sh.at[pl.ds(s * 51200 + i * 2048, 2048)])
        return _
    lax.fori_loop(0, 25, zero_m, None)
    plsc.subcore_barrier()

    pltpu.sync_copy(tval_hbm.at[pl.ds(c * 8 * 16, 128)], tv_all)
    pltpu.sync_copy(ibnd_hbm.at[pl.ds(c * 8 * 16, 128)], ib_all)

    for p in range(2):
        for t in range(6):
            sidxb[p][pl.ds(4000 + t * 16, 16)] = jnp.full((16,), DUMMY,
                                                          _i32)
            svalb[p][pl.ds(4000 + t * 16, 16)] = zi

    WIN = 2000
    NWINM = NN // WIN

    for p in range(2):
        pltpu.async_copy(
            keys_hbm.at[pl.ds((c * 8 + p) * E + s * NN, WIN)],
            keyb[p], skins[p])

    def win(wi, _):
        eoff = s * NN + wi * WIN
        pltpu.sync_copy(src_hbm.at[pl.ds(eoff, WIN)], src_v)
        pltpu.sync_copy(dst_hbm.at[pl.ds(eoff, WIN)], dst_v)
        for bl in range(8):
            step = wi * 8 + bl
            p = bl % 2
            bg = c * 8 + bl
            pltpu.make_async_copy(
                keys_hbm.at[pl.ds(bg * E + eoff, WIN)],
                keyb[p], skins[p]).wait()

            @pl.when(step >= 2)
            def _wait_scat():
                pltpu.make_async_copy(svalb[p], marks_sh.at[sidxb[p]],
                                      sscats[p]).wait()
            tv = tv_all[pl.ds(bl * 16, 16)]
            iv = ib_all[pl.ds(bl * 16, 16)]

            def inner(i, _):
                for j in range(5):
                    pos = i * 80 + j * 16
                    k16 = keyb[p][pl.ds(pos, 16)]
                    s16 = src_v[pl.ds(pos, 16)]
                    d16 = dst_v[pl.ds(pos, 16)]
                    gi16 = eoff + pos + lane
                    sel = (k16 > tv) | ((k16 == tv) & (gi16 <= iv))
                    val = sel.astype(_i32)
                    sidxb[p][pl.ds(pos, 16)] = bl * ROWS + s16
                    svalb[p][pl.ds(pos, 16)] = val
                    sidxb[p][pl.ds(2000 + pos, 16)] = bl * ROWS + d16
                    svalb[p][pl.ds(2000 + pos, 16)] = val
                return _
            lax.fori_loop(0, WIN // 80, inner, None)
            pltpu.async_copy(svalb[p], marks_sh.at[sidxb[p]], sscats[p],
                             add=True)

            bl2 = (bl + 2) % 8
            wi2 = wi + (bl + 2) // 8
            bg2 = c * 8 + bl2

            @pl.when(wi2 < NWINM)
            def _prefetch():
                eoff2 = s * NN + wi2 * WIN
                pltpu.async_copy(
                    keys_hbm.at[pl.ds(bg2 * E + eoff2, WIN)],
                    keyb[p], skins[p])
        return _
    lax.fori_loop(0, NWINM, win, None)
    for p in range(2):
        pltpu.make_async_copy(svalb[p], marks_sh.at[sidxb[p]],
                              sscats[p]).wait()
    plsc.subcore_barrier()

    cnts = jnp.zeros((16,), _i32)
    for bl in range(8):
        pltpu.sync_copy(
            marks_sh.at[pl.ds(bl * ROWS + s * 6400, 6400)], seg_v)

        def scan(i, ptr):
            m16 = seg_v[pl.ds(i * 16, 16)] > 0
            gid16 = s * 6400 + i * 16 + lane
            m16 = m16 & (gid16 < NN)
            cnt = jnp.sum(m16.astype(_i32))

            @pl.when(ptr < 128)
            def _store():
                plsc.store_compressed(ids_v.at[pl.ds(ptr, 16)], gid16,
                                      mask=m16)
            return ptr + cnt
        ptr = lax.fori_loop(0, 400, scan, _i32(0))
        cnts = jnp.where(lane == bl, ptr, cnts)
        pltpu.sync_copy(ids_v, coll_sh.at[pl.ds((s * 8 + bl) * 160, 160)])
    cnt_v[...] = cnts
    pltpu.sync_copy(cnt_v, cnts_sh.at[pl.ds(s * 16, 16)])
    plsc.subcore_barrier()

    @pl.when(s < 8)
    def _assemble():
        pltpu.sync_copy(cnts_sh, call_v)
        for seg in range(16):
            pltpu.sync_copy(coll_sh.at[pl.ds((seg * 8 + s) * 160, 160)],
                            coll_v.at[pl.ds(seg * 160, 160)])
        counts16 = plsc.load_gather(call_v, [lane * 16 + s])
        capped = jnp.minimum(counts16, 128)
        exclc = plsc.cumsum(capped) - capped
        take = jnp.clip(128 - exclc, 0, capped)
        opos = plsc.cumsum(take) - take
        bound = plsc.cumsum(take)
        total = jnp.sum(take)
        sb_v[pl.ds(0, 16)] = bound
        sb_v[pl.ds(16, 16)] = opos

        for j in range(8):
            p16 = j * 16 + lane
            segidx = jnp.zeros((16,), _i32)
            for t in range(16):
                bt = plsc.load_gather(sb_v, [jnp.full((16,), t, _i32)])
                segidx = segidx + (bt <= p16).astype(_i32)
            segidx = jnp.minimum(segidx, 15)
            op = plsc.load_gather(sb_v, [16 + segidx])
            addr = segidx * 160 + (p16 - op)
            ids16 = plsc.load_gather(coll_v, [addr])
            out_v[pl.ds(j * 16, 16)] = jnp.where(p16 < total, ids16, 0)
        bg = c * 8 + s
        pltpu.sync_copy(out_v, nodes_hbm.at[pl.ds(bg * 128, 128)])


def _mark_scan(keys, src, dst, tval, ibnd):
    f = pl.kernel(
        _mark_body,
        out_type=jax.ShapeDtypeStruct((NQ * 128,), _i32),
        mesh=_mesh,
        compiler_params=pltpu.CompilerParams(needs_layout_passes=False),
        scratch_types=[
            pltpu.VMEM_SHARED((MARKS,), _i32),
            pltpu.VMEM_SHARED((16 * 8 * 160,), _i32),
            pltpu.VMEM_SHARED((256,), _i32),
            pltpu.VMEM((2000,), _i32),
            pltpu.VMEM((2000,), _i32),
            pltpu.VMEM((2000,), _u32),
            pltpu.VMEM((2000,), _u32),
            pltpu.VMEM((4096,), _i32),
            pltpu.VMEM((4096,), _i32),
            pltpu.VMEM((4096,), _i32),
            pltpu.VMEM((4096,), _i32),
            pltpu.VMEM((2048,), _i32),
            pltpu.VMEM((128,), _u32),
            pltpu.VMEM((128,), _i32),
            pltpu.VMEM((6400,), _i32),
            pltpu.VMEM((160,), _i32),
            pltpu.VMEM((16,), _i32),
            pltpu.VMEM((256,), _i32),
            pltpu.VMEM((16 * 160,), _i32),
            pltpu.VMEM((32,), _i32),
            pltpu.VMEM((128,), _i32),
            pltpu.SemaphoreType.DMA,
            pltpu.SemaphoreType.DMA,
            pltpu.SemaphoreType.DMA,
            pltpu.SemaphoreType.DMA,
        ],
    )
    return f(keys, src, dst, tval, ibnd)


def _gather_body(table_hbm, idx_hbm, out_hbm, idx_v, rows_v, sem):
    wid = _wid()
    base = wid * 64
    pltpu.sync_copy(idx_hbm.at[pl.ds(base, 64)], idx_v)
    pltpu.async_copy(table_hbm.at[idx_v], rows_v, sem).wait()
    pltpu.sync_copy(rows_v, out_hbm.at[pl.ds(base, 64)])


def _gather_rows(table, idx):
    f = pl.kernel(
        _gather_body,
        out_type=jax.ShapeDtypeStruct((NQ * 128, DF), jnp.float32),
        mesh=_mesh,
        compiler_params=pltpu.CompilerParams(needs_layout_passes=False),
        scratch_types=[
            pltpu.VMEM((64,), _i32),
            pltpu.VMEM((64, DF), jnp.float32),
            pltpu.SemaphoreType.DMA,
        ],
    )
    return f(table, idx)


def _rep16(x, dtype):
    return jnp.tile(x.astype(dtype)[:, None], (1, 16)).reshape(-1)


@jax.jit
def _run(attention_question, question_entities, edge_index, edge_weights,
         node_table, w_imp, num_max_nodes):
    importance = jax.nn.sigmoid(attention_question * w_imp)
    contrib = importance * (importance >= 0.5).astype(importance.dtype)

    ents = jnp.pad(question_entities, ((0, 0), (0, 12))).reshape(-1)
    cons = jnp.pad(contrib, ((0, 0), (0, 12))).reshape(-1)
    src = edge_index[0]
    dst = edge_index[1]

    keys = _keygen(ents, cons, src, dst, edge_weights)
    if _STOP == 1:
        return keys[:NQ * 128 * DF].astype(jnp.float32).reshape(NQ, 128, DF)

    zero16 = jnp.zeros((NQ * 16,), _i32)
    k1 = jnp.full((NQ,), KSEL, _i32)
    h1 = _hist_p1(keys, _rep16(jnp.zeros((NQ,), _u32), _u32), zero16)
    i1, k2 = _pick_desc(h1, k1)
    h2 = _hist_p2(keys, _rep16(i1, _u32), zero16)
    i2, k3 = _pick_desc(h2, k2)
    h3 = _hist_p3(keys, _rep16((i1 << 11) | i2, _u32), zero16)
    i3, r = _pick_desc(h3, k3)
    tval = ((i1.astype(_u32) << 20) | (i2.astype(_u32) << 9)
            | i3.astype(_u32))
    ha = _hist_tA(keys, _rep16(tval, _u32), zero16)
    ia, rb = _pick_asc(ha, r)
    hb = _hist_tB(keys, _rep16(tval, _u32), _rep16(ia, _i32))
    ib, _ = _pick_asc(hb, rb)
    ibnd = ia * 1024 + ib
    if _STOP == 2:
        return (jnp.zeros((NQ, 128, DF), jnp.float32)
                + (tval.sum() + ibnd.sum()).astype(jnp.float32))

    nodes = _mark_scan(keys, src, dst, _rep16(tval, _u32),
                       _rep16(ibnd, _i32))
    nodes = nodes + (jnp.asarray(num_max_nodes, _i32) - 128)
    out = _gather_rows(node_table, nodes)
    return out.reshape(NQ, 128, DF)


def kernel(attention_question, question_entities, edge_index, edge_weights,
           node_table, w_imp, num_max_nodes):
    return _run(attention_question, question_entities, edge_index,
                edge_weights, node_table, w_imp, num_max_nodes)
